# baseline (speedup 1.0000x reference)
import functools

import jax
import jax.numpy as jnp
from jax import lax
from jax.experimental import pallas as pl
from jax.experimental.pallas import tpu as pltpu
from jax.experimental.pallas import tpu_sc as plsc

_B, _C, _N, _D, _L = 4, 4, 1000, 64, 12
_RB = 200
_NRB = _N // _RB
_BETA = 0.02
_INV_SQRT10 = 10.0 ** -0.5
_K = 5

_NW = 32
_IPW = 504
_ITEMS = _IPW * _NW
_RPW = _IPW * _K
_CHI = 24
_CH = _CHI * _K
_NCH = _IPW // _CHI
_TW = 128


def _fused_tc_kernel(x_ref, a_ref, m_ref, am_ref, idx_ref):
    rb = pl.program_id(1)
    row0 = rb * _RB

    s_sum = jnp.zeros((_RB, _N), jnp.float32)
    for c in range(_C):
        xr = x_ref[0, c, pl.ds(row0, _RB), :].astype(jnp.bfloat16)
        xc = x_ref[0, c].astype(jnp.bfloat16)
        logits = lax.dot_general(xr, xc, (((1,), (1,)), ((), ())),
                                 preferred_element_type=jnp.float32)
        mx = jnp.max(logits, axis=-1, keepdims=True)
        e = jnp.exp(logits - mx)
        s_sum = s_sum + e / jnp.sum(e, axis=-1, keepdims=True)

    a_base = (a_ref[0, 0] + a_ref[0, 1] + a_ref[0, 2] + a_ref[0, 3]) * (1.0 / _C) \
        + (_BETA / 2)
    z_scale = 0.5 * _INV_SQRT10 / _C

    aug = jnp.where(
        lax.broadcasted_iota(jnp.int32, (_N, 2), 1) == 0,
        jnp.float32(1.0),
        lax.broadcasted_iota(jnp.int32, (_N, 2), 0).astype(jnp.float32))

    for c in range(_C):
        mc = m_ref[0, c]
        mr = m_ref[0, c, pl.ds(row0, _RB), :]
        prod = lax.dot_general(mr.astype(jnp.bfloat16), mc.astype(jnp.bfloat16),
                               (((1,), (1,)), ((), ())),
                               preferred_element_type=jnp.float32)
        am_c = a_base + (_BETA / 2) * jnp.tanh(s_sum * prod * z_scale)
        am_ref[0, c] = am_c

        keys = am_c
        cols = []
        for j in range(_K):
            kmax = jnp.max(keys, axis=-1, keepdims=True)
            sel = keys == kmax
            row = lax.dot_general(sel.astype(jnp.float32), aug,
                                  (((1,), (0,)), ((), ())),
                                  precision=lax.Precision.HIGHEST,
                                  preferred_element_type=jnp.float32)
            loc = jnp.clip(jnp.floor(row[:, 1:] / row[:, :1] + 0.5),
                           0.0, float(_N - 1))
            cols.append(loc)
            if j < _K - 1:
                keys = jnp.where(sel, -jnp.inf, keys)
        off = (pl.program_id(0) * _C + c) * _N
        idx_ref[0, c] = jnp.concatenate(cols, axis=1).astype(jnp.int32) + off


def _sc_gather_max(table, idx):
    mesh = plsc.VectorSubcoreMesh(core_axis_name="c", subcore_axis_name="s")

    @functools.partial(
        pl.kernel, mesh=mesh,
        out_type=jax.ShapeDtypeStruct((_ITEMS, 16), jnp.float32),
        scratch_types=[
            pltpu.VMEM((_RPW,), jnp.int32),
            pltpu.VMEM((_CH, _TW), jnp.float32),
            pltpu.VMEM((_CH, _TW), jnp.float32),
            pltpu.VMEM((_IPW, 16), jnp.float32),
            pltpu.SemaphoreType.DMA,
            pltpu.SemaphoreType.DMA,
        ],
    )
    def k(table_hbm, idx_hbm, out_hbm, idx_v, buf0, buf1, out_v, sem0, sem1):
        wid = lax.axis_index("s") * 2 + lax.axis_index("c")
        pltpu.sync_copy(idx_hbm.at[pl.ds(wid * _RPW, _RPW)], idx_v)
        bufs, sems = (buf0, buf1), (sem0, sem1)

        def issue(g):
            return pltpu.async_copy(
                table_hbm.at[idx_v.at[pl.ds(g * _CH, _CH)]],
                bufs[g % 2], sems[g % 2])

        pending = issue(0)
        for g in range(_NCH):
            if g + 1 < _NCH:
                nxt = issue(g + 1)
            pending.wait()
            buf = bufs[g % 2]
            for ii in range(_CHI):
                r = ii * _K
                m = jnp.maximum(buf[r, pl.ds(0, 16)], buf[r + 1, pl.ds(0, 16)])
                m = jnp.maximum(m, buf[r + 2, pl.ds(0, 16)])
                m = jnp.maximum(m, buf[r + 3, pl.ds(0, 16)])
                m = jnp.maximum(m, buf[r + 4, pl.ds(0, 16)])
                out_v[g * _CHI + ii] = m
            if g + 1 < _NCH:
                pending = nxt
        pltpu.sync_copy(out_v, out_hbm.at[pl.ds(wid * _IPW, _IPW)])

    return k(table, idx)


def kernel(x, A, mask, k):
    del k
    grid = (_B, _NRB)
    am, idxs = pl.pallas_call(
        _fused_tc_kernel,
        grid=grid,
        in_specs=[
            pl.BlockSpec((1, _C, _N, _D), lambda b, r: (b, 0, 0, 0)),
            pl.BlockSpec((1, _C, _RB, _N), lambda b, r: (b, 0, r, 0)),
            pl.BlockSpec((1, _C, _N, _L), lambda b, r: (b, 0, 0, 0)),
        ],
        out_specs=[
            pl.BlockSpec((1, _C, _RB, _N), lambda b, r: (b, 0, r, 0)),
            pl.BlockSpec((1, _C, _RB, _K), lambda b, r: (b, 0, r, 0)),
        ],
        out_shape=[
            jax.ShapeDtypeStruct((_B, _C, _N, _N), jnp.float32),
            jax.ShapeDtypeStruct((_B, _C, _N, _K), jnp.int32),
        ],
        compiler_params=pltpu.CompilerParams(
            dimension_semantics=("parallel", "parallel"),
        ),
    )(x, A, mask)

    nrows = _B * _C * _N
    table = jnp.pad(mask.reshape(nrows, _L),
                    ((0, _ITEMS - nrows), (0, _TW - _L)))
    idx_flat = jnp.pad(idxs.reshape(-1), (0, _ITEMS * _K - nrows * _K))
    out16 = _sc_gather_max(table, idx_flat)
    mu = out16[:nrows, :_L].reshape(_B, _C, _N, _L)
    return (am, mu)


# speedup vs baseline: 3.3805x; 3.3805x over previous
import jax
import jax.numpy as jnp
from jax import lax
from jax.experimental import pallas as pl
from jax.experimental.pallas import tpu as pltpu

_B, _C, _N, _D, _L = 4, 4, 1000, 64, 12
_RB = 200
_NRB = _N // _RB
_BETA = 0.02
_INV_SQRT10 = 10.0 ** -0.5
_K = 5


def _fused_kernel(x_ref, a_ref, m_ref, am_ref, mu_ref):
    rb = pl.program_id(1)
    row0 = rb * _RB

    s_sum = jnp.zeros((_RB, _N), jnp.float32)
    for c in range(_C):
        xr = x_ref[0, c, pl.ds(row0, _RB), :]
        xc = x_ref[0, c]
        logits = lax.dot_general(xr, xc, (((1,), (1,)), ((), ())),
                                 preferred_element_type=jnp.float32)
        mx = jnp.max(logits, axis=-1, keepdims=True)
        e = jnp.exp(logits - mx)
        s_sum = s_sum + e / jnp.sum(e, axis=-1, keepdims=True)

    a_base = (a_ref[0, 0] + a_ref[0, 1] + a_ref[0, 2] + a_ref[0, 3]) * (1.0 / _C) \
        + (_BETA / 2)
    z_scale = 0.5 * _INV_SQRT10 / _C

    for c in range(_C):
        mc = m_ref[0, c]
        mr = m_ref[0, c, pl.ds(row0, _RB), :]
        prod = lax.dot_general(mr, mc, (((1,), (1,)), ((), ())),
                               preferred_element_type=jnp.float32)
        am_c = a_base + (_BETA / 2) * jnp.tanh(s_sum * prod * z_scale)
        am_ref[0, c] = am_c

        mc_aug = jnp.concatenate(
            [mc, jnp.ones((_N, 1), jnp.float32)], axis=1)
        keys = am_c
        best = jnp.full((_RB, _L), -jnp.inf, jnp.float32)
        for j in range(_K):
            kmax = jnp.max(keys, axis=-1, keepdims=True)
            sel = keys == kmax
            row = lax.dot_general(sel.astype(jnp.float32), mc_aug,
                                  (((1,), (0,)), ((), ())),
                                  preferred_element_type=jnp.float32)
            best = jnp.maximum(best, row[:, :_L] / row[:, _L:])
            if j < _K - 1:
                keys = jnp.where(sel, -jnp.inf, keys)
        mu_ref[0, c] = best


def kernel(x, A, mask, k):
    del k
    grid = (_B, _NRB)
    am, mu = pl.pallas_call(
        _fused_kernel,
        grid=grid,
        in_specs=[
            pl.BlockSpec((1, _C, _N, _D), lambda b, r: (b, 0, 0, 0)),
            pl.BlockSpec((1, _C, _RB, _N), lambda b, r: (b, 0, r, 0)),
            pl.BlockSpec((1, _C, _N, _L), lambda b, r: (b, 0, 0, 0)),
        ],
        out_specs=[
            pl.BlockSpec((1, _C, _RB, _N), lambda b, r: (b, 0, r, 0)),
            pl.BlockSpec((1, _C, _RB, _L), lambda b, r: (b, 0, r, 0)),
        ],
        out_shape=[
            jax.ShapeDtypeStruct((_B, _C, _N, _N), jnp.float32),
            jax.ShapeDtypeStruct((_B, _C, _N, _L), jnp.float32),
        ],
        compiler_params=pltpu.CompilerParams(
            dimension_semantics=("parallel", "parallel"),
        ),
    )(x, A, mask)
    return (am, mu)
